# baseline (speedup 1.0000x reference)
import jax
import jax.numpy as jnp
from jax import lax
from jax.experimental import pallas as pl
from jax.experimental.pallas import tpu as pltpu
from jax.experimental.pallas import tpu_sc as plsc

_NC = 2
_NS = 16
_NW = _NC * _NS
_LANES = 16

_B = 16384
_L = 200
_VOCAB = 1000000

_ROWS_PER_W = _B // _NW
_GROUPS = _ROWS_PER_W // _LANES
_G_IDX = _LANES * _L
_SPLIT = 128
_NSPLIT = _G_IDX // _SPLIT


def _sc_body(batch_hbm, w_hbm, b_hbm, out_hbm, idx2d_v, idxf_v, val_v,
             out_v, b_v, w_sh, idx_sem, val_sem):
    sid = lax.axis_index("s")
    wid = sid * _NC + lax.axis_index("c")
    row0 = wid * _ROWS_PER_W

    pltpu.sync_copy(b_hbm, b_v)
    b_vec = b_v[...]
    iota = lax.iota(jnp.int32, _LANES)
    lane2 = 2 * iota

    def idx_copy(g, buf):
        return pltpu.make_async_copy(
            batch_hbm.at[pl.ds(row0 + g * _LANES, _LANES), :],
            idx2d_v.at[buf], idx_sem)

    def relayout(buf):
        base = buf * _G_IDX

        @pl.loop(0, _L, unroll=4)
        def _(j):
            t = plsc.load_gather(idx2d_v.at[buf], [iota, jnp.full((_LANES,), 0, jnp.int32) + j])
            plsc.store_scatter(idxf_v, [base + _LANES * j + iota], t)

    def val_copies(buf):
        cps = []
        for c in range(_NSPLIT):
            off = pl.multiple_of(buf * _G_IDX + c * _SPLIT, _SPLIT)
            cps.append(pltpu.make_async_copy(
                w_sh.at[idxf_v.at[pl.ds(off, _SPLIT)]],
                val_v.at[pl.ds(off, _SPLIT)], val_sem))
        return cps

    idx_copy(0, 0).start()

    @pl.when(sid == 0)
    def _():
        pltpu.sync_copy(w_hbm, w_sh)
    plsc.subcore_barrier()

    idx_copy(0, 0).wait()
    relayout(0)
    for cp in val_copies(0):
        cp.start()
    idx_copy(1, 1).start()

    @pl.loop(0, _GROUPS)
    def _group(g):
        buf = g % 2
        nbuf = 1 - buf
        for cp in val_copies(buf):
            cp.wait()
        g2 = jnp.minimum(g + 2, _GROUPS - 1)
        idx_copy(g2, buf).start()
        idx_copy(g, nbuf).wait()
        relayout(nbuf)
        for cp in val_copies(nbuf):
            cp.start()

        vbase = buf * _G_IDX

        def red_body(j, accs):
            a0, a1 = accs
            base = vbase + j * (8 * _LANES) + iota
            for k in range(0, 8, 2):
                a0 = a0 + plsc.load_gather(val_v, [base + k * _LANES])
                a1 = a1 + plsc.load_gather(val_v, [base + (k + 1) * _LANES])
            return a0, a1

        acc0, acc1 = lax.fori_loop(0, _L // 8, red_body, (b_vec, b_vec * 0.0))
        x = acc0 + acc1
        pos = 1.0 / (1.0 + jnp.exp(-x))
        neg = 1.0 / (1.0 + jnp.exp(x))
        slot = g * (2 * _LANES) + lane2
        plsc.store_scatter(out_v, [slot], pos)
        plsc.store_scatter(out_v, [slot + 1], neg)

    for cp in val_copies(0):
        cp.wait()
    idx_copy(_GROUPS - 1, 1).wait()

    pltpu.sync_copy(
        out_v, out_hbm.at[pl.ds(wid * (2 * _ROWS_PER_W), 2 * _ROWS_PER_W)])


def kernel(batch, w, b):
    w_flat = w.reshape(-1)
    b16 = jnp.broadcast_to(b, (_LANES,)).astype(jnp.float32)

    mesh = plsc.VectorSubcoreMesh(core_axis_name="c", subcore_axis_name="s")
    out_flat = pl.kernel(
        _sc_body,
        out_type=jax.ShapeDtypeStruct((_B * 2,), jnp.float32),
        mesh=mesh,
        scratch_types=[
            pltpu.VMEM((2, _LANES, _L), jnp.int32),
            pltpu.VMEM((2 * _G_IDX,), jnp.int32),
            pltpu.VMEM((2 * _G_IDX,), jnp.float32),
            pltpu.VMEM((2 * _ROWS_PER_W,), jnp.float32),
            pltpu.VMEM((_LANES,), jnp.float32),
            pltpu.VMEM_SHARED((_VOCAB,), jnp.float32),
            pltpu.SemaphoreType.DMA,
            pltpu.SemaphoreType.DMA,
        ],
        compiler_params=pltpu.CompilerParams(needs_layout_passes=False),
    )(batch, w_flat, b16)
    return out_flat.reshape(_B, 2)


# speedup vs baseline: 1.4625x; 1.4625x over previous
import jax
import jax.numpy as jnp
from jax import lax
from jax.experimental import pallas as pl
from jax.experimental.pallas import tpu as pltpu
from jax.experimental.pallas import tpu_sc as plsc

_NC = 2
_NS = 16
_NW = _NC * _NS
_LANES = 16

_B = 16384
_L = 200
_VOCAB = 1000000

_ROWS_PER_W = _B // _NW
_GROUPS = _ROWS_PER_W // _LANES
_G_IDX = _LANES * _L
_SPLIT = 128
_NSPLIT = _G_IDX // _SPLIT


def _sc_body(idx_hbm, w_hbm, b_hbm, out_hbm, idx_v, val_v, out_v, b_v, w_sh,
             idx_sem, val_sem):
    sid = lax.axis_index("s")
    wid = sid * _NC + lax.axis_index("c")

    pltpu.sync_copy(b_hbm, b_v)
    b_vec = b_v[...]
    lane_base = lax.iota(jnp.int32, _LANES) * _L
    lane2 = 2 * lax.iota(jnp.int32, _LANES)

    def idx_copy(g, buf):
        src = pl.multiple_of((wid * _GROUPS + g) * _G_IDX, _SPLIT)
        dst = pl.multiple_of(buf * _G_IDX, _SPLIT)
        return pltpu.make_async_copy(
            idx_hbm.at[pl.ds(src, _G_IDX)],
            idx_v.at[pl.ds(dst, _G_IDX)], idx_sem)

    def val_copies(buf):
        cps = []
        for c in range(_NSPLIT):
            off = pl.multiple_of(buf * _G_IDX + c * _SPLIT, _SPLIT)
            cps.append(pltpu.make_async_copy(
                w_sh.at[idx_v.at[pl.ds(off, _SPLIT)]],
                val_v.at[pl.ds(off, _SPLIT)], val_sem))
        return cps

    idx_copy(0, 0).start()

    @pl.when(sid == 0)
    def _():
        pltpu.sync_copy(w_hbm, w_sh)
    plsc.subcore_barrier()

    idx_copy(0, 0).wait()
    for cp in val_copies(0):
        cp.start()
    idx_copy(1, 1).start()

    @pl.loop(0, _GROUPS)
    def _group(g):
        buf = g % 2
        nbuf = 1 - buf
        for cp in val_copies(buf):
            cp.wait()
        g2 = jnp.minimum(g + 2, _GROUPS - 1)
        idx_copy(g2, buf).start()
        idx_copy(g, nbuf).wait()
        for cp in val_copies(nbuf):
            cp.start()

        vbase = buf * _G_IDX + lane_base

        def red_body(j, accs):
            a0, a1 = accs
            base = vbase + j * 8
            for k in range(0, 8, 2):
                a0 = a0 + plsc.load_gather(val_v, [base + k])
                a1 = a1 + plsc.load_gather(val_v, [base + k + 1])
            return a0, a1

        acc0, acc1 = lax.fori_loop(0, _L // 8, red_body, (b_vec, b_vec * 0.0))
        x = acc0 + acc1
        pos = 1.0 / (1.0 + jnp.exp(-x))
        neg = 1.0 / (1.0 + jnp.exp(x))
        slot = g * (2 * _LANES) + lane2
        plsc.store_scatter(out_v, [slot], pos)
        plsc.store_scatter(out_v, [slot + 1], neg)

    for cp in val_copies(0):
        cp.wait()
    idx_copy(_GROUPS - 1, 1).wait()

    pltpu.sync_copy(
        out_v, out_hbm.at[pl.ds(wid * (2 * _ROWS_PER_W), 2 * _ROWS_PER_W)])


def kernel(batch, w, b):
    idx_flat = batch.reshape(-1)
    w_flat = w.reshape(-1)
    b16 = jnp.broadcast_to(b, (_LANES,)).astype(jnp.float32)

    mesh = plsc.VectorSubcoreMesh(core_axis_name="c", subcore_axis_name="s")
    out_flat = pl.kernel(
        _sc_body,
        out_type=jax.ShapeDtypeStruct((_B * 2,), jnp.float32),
        mesh=mesh,
        scratch_types=[
            pltpu.VMEM((2 * _G_IDX,), jnp.int32),
            pltpu.VMEM((2 * _G_IDX,), jnp.float32),
            pltpu.VMEM((2 * _ROWS_PER_W,), jnp.float32),
            pltpu.VMEM((_LANES,), jnp.float32),
            pltpu.VMEM_SHARED((_VOCAB,), jnp.float32),
            pltpu.SemaphoreType.DMA,
            pltpu.SemaphoreType.DMA,
        ],
        compiler_params=pltpu.CompilerParams(needs_layout_passes=False),
    )(idx_flat, w_flat, b16)
    return out_flat.reshape(_B, 2)
